# baseline (speedup 1.0000x reference)
import functools

import jax
import jax.numpy as jnp
from jax import lax
from jax.experimental import pallas as pl
from jax.experimental.pallas import tpu as pltpu
from jax.experimental.pallas import tpu_sc as plsc

NC = 2
NS = 16
NW = NC * NS
CH = 128
NBUF = 3

_SC_PARAMS = pltpu.CompilerParams(use_tc_tiling_on_sc=False)


def _zchunks(zrows):
    out = []
    off = 0
    while off < zrows:
        out.append((off, min(CH, zrows - off)))
        off += CH
    return out


def _fill(ref, nrows, d, vec16):
    def fi(i, carry):
        for jj in range(d // 16):
            ref[i, pl.ds(jj * 16, 16)] = vec16
        return carry
    lax.fori_loop(0, nrows, fi, 0)


def _seg_kernel(n_pad, nchunk, d, with_deg):
    zrows = n_pad // NS
    base = nchunk // NW
    left = nchunk - base * NW
    ngroups = base // NBUF
    assert base == ngroups * NBUF
    zck = _zchunks(zrows)
    mesh = plsc.VectorSubcoreMesh(core_axis_name="c", subcore_axis_name="s")

    out_type = [jax.ShapeDtypeStruct((NC, n_pad, d), jnp.float32)]
    scratch = [
        pltpu.VMEM_SHARED((n_pad, d), jnp.float32),
        pltpu.VMEM_SHARED((n_pad, d), jnp.float32),
        pltpu.VMEM((base + 1, CH), jnp.int32),
        pltpu.VMEM((base + 1, CH), jnp.int32),
    ]
    scratch += [pltpu.VMEM((CH, d), jnp.float32) for _ in range(NBUF)]
    scratch += [pltpu.SemaphoreType.DMA for _ in range(NBUF)]
    if with_deg:
        out_type.append(jax.ShapeDtypeStruct((NC * n_pad,), jnp.float32))
        scratch.append(pltpu.VMEM_SHARED((n_pad,), jnp.float32))
        scratch.append(pltpu.VMEM((CH,), jnp.float32))
        scratch.append(pltpu.VMEM((zrows,), jnp.float32))
        scratch.append(pltpu.SemaphoreType.DMA)

    def body(y_h, ei_h, part_h, *rest):
        if with_deg:
            deg_h = rest[0]
            rest = rest[1:]
        ycp, acc, si2, di2 = rest[:4]
        rows = rest[4:4 + NBUF]
        gsem = rest[4 + NBUF:4 + 2 * NBUF]
        if with_deg:
            dacc, ones, dstg, dsem = rest[4 + 2 * NBUF:]
        c = lax.axis_index("c")
        s = lax.axis_index("s")
        wid = s * NC + c
        r0 = s * zrows
        row0 = wid * base
        xrow = jnp.minimum(NW * base + wid, nchunk - 1)
        pltpu.sync_copy(ei_h.at[0, pl.ds(row0, base)],
                        si2.at[pl.ds(0, base)])
        pltpu.sync_copy(ei_h.at[1, pl.ds(row0, base)],
                        di2.at[pl.ds(0, base)])
        pltpu.sync_copy(ei_h.at[0, xrow], si2.at[base])
        pltpu.sync_copy(ei_h.at[1, xrow], di2.at[base])
        _fill(rows[1], CH, d, jnp.zeros((16,), jnp.float32))
        for zo, zs in zck:
            pltpu.sync_copy(y_h.at[pl.ds(r0 + zo, zs)],
                            rows[0].at[pl.ds(0, zs)])
            pltpu.sync_copy(rows[0].at[pl.ds(0, zs)],
                            ycp.at[pl.ds(r0 + zo, zs)])
            pltpu.sync_copy(rows[1].at[pl.ds(0, zs)],
                            acc.at[pl.ds(r0 + zo, zs)])
        if with_deg:
            for i in range(CH // 16):
                ones[pl.ds(i * 16, 16)] = jnp.ones((16,), jnp.float32)
            zd = jnp.zeros((16,), jnp.float32)
            for i in range(zrows // 16):
                dstg[pl.ds(i * 16, 16)] = zd
            dstg[pl.ds(zrows - 16, 16)] = zd
            pltpu.sync_copy(dstg, dacc.at[pl.ds(r0, zrows)])
        plsc.subcore_barrier()

        pltpu.async_copy(ycp.at[si2.at[0]], rows[0], gsem[0])
        pltpu.async_copy(ycp.at[si2.at[1]], rows[1], gsem[1])

        def deg_wait():
            pltpu.make_async_copy(ones, dacc.at[di2.at[0]], dsem).wait()

        def group(g, carry):
            for b in range(NBUF):
                j = g * NBUF + b
                q = (b + 2) % NBUF
                pltpu.make_async_copy(ycp.at[si2.at[b]], rows[b],
                                      gsem[b]).wait()
                if b == 0:
                    pltpu.async_copy(ycp.at[si2.at[j + 2]], rows[q], gsem[q])
                else:
                    @pl.when(g < ngroups - 1)
                    def _():
                        pltpu.async_copy(ycp.at[si2.at[j + 2]], rows[q],
                                         gsem[q])
                pltpu.sync_copy(rows[b], acc.at[di2.at[j]], add=True)
                if with_deg:
                    pltpu.async_copy(ones, dacc.at[di2.at[j]], dsem,
                                     add=True)

                    @pl.when(g >= 2)
                    def _():
                        deg_wait()
            return carry

        lax.fori_loop(0, ngroups, group, 0)
        if with_deg:
            for _ in range(2 * NBUF):
                deg_wait()

        @pl.when(wid < left)
        def _():
            pltpu.sync_copy(ycp.at[si2.at[base]], rows[0])
            pltpu.sync_copy(rows[0], acc.at[di2.at[base]], add=True)
            if with_deg:
                pltpu.sync_copy(ones, dacc.at[di2.at[base]], add=True)

        plsc.subcore_barrier()
        for zo, zs in zck:
            pltpu.sync_copy(acc.at[pl.ds(r0 + zo, zs)],
                            rows[0].at[pl.ds(0, zs)])
            pltpu.sync_copy(rows[0].at[pl.ds(0, zs)],
                            part_h.at[c, pl.ds(r0 + zo, zs)])
        if with_deg:
            pltpu.sync_copy(dacc.at[pl.ds(r0, zrows)], dstg)
            pltpu.sync_copy(dstg, deg_h.at[pl.ds(c * n_pad + r0, zrows)])

    return pl.kernel(body, out_type=tuple(out_type), mesh=mesh,
                     scratch_types=scratch, compiler_params=_SC_PARAMS)


def _seg1_kernel(n_pad, nchunk):
    zrows = n_pad // NS
    base = nchunk // NW
    left = nchunk - base * NW
    ngroups = base // NBUF
    mesh = plsc.VectorSubcoreMesh(core_axis_name="c", subcore_axis_name="s")

    scratch = [
        pltpu.VMEM_SHARED((n_pad,), jnp.float32),
        pltpu.VMEM_SHARED((n_pad,), jnp.float32),
        pltpu.VMEM((base + 1, CH), jnp.int32),
        pltpu.VMEM((base + 1, CH), jnp.int32),
        pltpu.VMEM((zrows,), jnp.float32),
    ]
    scratch += [pltpu.VMEM((CH,), jnp.float32) for _ in range(NBUF)]
    scratch += [pltpu.SemaphoreType.DMA for _ in range(NBUF)]

    def body(y_h, ei_h, part_h, ycp, acc, si2, di2, stg, *rest):
        vals = rest[:NBUF]
        gsem = rest[NBUF:]
        c = lax.axis_index("c")
        s = lax.axis_index("s")
        wid = s * NC + c
        r0 = s * zrows
        row0 = wid * base
        xrow = jnp.minimum(NW * base + wid, nchunk - 1)
        pltpu.sync_copy(ei_h.at[0, pl.ds(row0, base)],
                        si2.at[pl.ds(0, base)])
        pltpu.sync_copy(ei_h.at[1, pl.ds(row0, base)],
                        di2.at[pl.ds(0, base)])
        pltpu.sync_copy(ei_h.at[0, xrow], si2.at[base])
        pltpu.sync_copy(ei_h.at[1, xrow], di2.at[base])
        pltpu.sync_copy(y_h.at[pl.ds(r0, zrows)], stg)
        pltpu.sync_copy(stg, ycp.at[pl.ds(r0, zrows)])
        zd = jnp.zeros((16,), jnp.float32)
        for i in range(zrows // 16):
            stg[pl.ds(i * 16, 16)] = zd
        stg[pl.ds(zrows - 16, 16)] = zd
        pltpu.sync_copy(stg, acc.at[pl.ds(r0, zrows)])
        plsc.subcore_barrier()

        pltpu.async_copy(ycp.at[si2.at[0]], vals[0], gsem[0])
        pltpu.async_copy(ycp.at[si2.at[1]], vals[1], gsem[1])

        def group(g, carry):
            for b in range(NBUF):
                j = g * NBUF + b
                q = (b + 2) % NBUF
                pltpu.make_async_copy(ycp.at[si2.at[b]], vals[b],
                                      gsem[b]).wait()
                if b == 0:
                    pltpu.async_copy(ycp.at[si2.at[j + 2]], vals[q], gsem[q])
                else:
                    @pl.when(g < ngroups - 1)
                    def _():
                        pltpu.async_copy(ycp.at[si2.at[j + 2]], vals[q],
                                         gsem[q])
                pltpu.sync_copy(vals[b], acc.at[di2.at[j]], add=True)
            return carry

        lax.fori_loop(0, ngroups, group, 0)

        @pl.when(wid < left)
        def _():
            pltpu.sync_copy(ycp.at[si2.at[base]], vals[0])
            pltpu.sync_copy(vals[0], acc.at[di2.at[base]], add=True)

        plsc.subcore_barrier()
        pltpu.sync_copy(acc.at[pl.ds(r0, zrows)], stg)
        pltpu.sync_copy(stg, part_h.at[pl.ds(c * n_pad + r0, zrows)])

    return pl.kernel(
        body,
        out_type=jax.ShapeDtypeStruct((NC * n_pad,), jnp.float32),
        mesh=mesh, scratch_types=scratch, compiler_params=_SC_PARAMS)


def _k1a_body(n, n_pad, x_ref, wl_ref, y_ref):
    y_ref[:n, :] = jnp.dot(x_ref[...], wl_ref[...],
                           preferred_element_type=jnp.float32)
    y_ref[n:, :] = jnp.zeros((n_pad - n, y_ref.shape[1]), jnp.float32)


def _k1b_body(x_ref, wr_ref, b_ref, r_ref):
    r_ref[...] = (jnp.dot(x_ref[...], wr_ref[...],
                          preferred_element_type=jnp.float32)
                  + b_ref[...][None, :])


def _k2_body(n, n_pad, p_ref, d_ref, r1_ref, g_ref, be_ref, wl_ref,
             x1_ref, y2_ref, inv_ref):
    agg = p_ref[0, :n, :] + p_ref[1, :n, :]
    deg = d_ref[0, :n] + d_ref[1, :n]
    inv = 1.0 / jnp.maximum(deg, 1.0)
    t = agg * inv[:, None] + r1_ref[...]
    mu = jnp.mean(t, axis=0)
    var = jnp.mean((t - mu[None, :]) ** 2, axis=0)
    xh = (t - mu[None, :]) * lax.rsqrt(var + 1e-5)[None, :]
    x1 = jnp.maximum(xh * g_ref[...][None, :] + be_ref[...][None, :], 0.0)
    x1_ref[...] = x1
    y2_ref[:n, :] = jnp.dot(x1, wl_ref[...],
                            preferred_element_type=jnp.float32)
    y2_ref[n:, :] = jnp.zeros((n_pad - n, y2_ref.shape[1]), jnp.float32)
    inv_ref[...] = inv


def _k2c_body(x1_ref, wr_ref, b2_ref, r2_ref):
    r2_ref[...] = (jnp.dot(x1_ref[...], wr_ref[...],
                           preferred_element_type=jnp.float32)
                   + b2_ref[...][None, :])


def _k3_body(n, n_pad, p_ref, inv_ref, x1_ref, r2_ref, g_ref, be_ref,
             w3l_ref, w3r_ref, b3_ref, y3_ref, r3_ref):
    agg = p_ref[0, :n, :] + p_ref[1, :n, :]
    inv = inv_ref[...]
    x1 = x1_ref[...]
    t = agg * inv[:, None] + r2_ref[...]
    mu = jnp.mean(t, axis=0)
    var = jnp.mean((t - mu[None, :]) ** 2, axis=0)
    xh = (t - mu[None, :]) * lax.rsqrt(var + 1e-5)[None, :]
    x2 = jnp.maximum(xh * g_ref[...][None, :] + be_ref[...][None, :], 0.0)
    x2 = x2 + x1
    y3_ref[:n] = jnp.sum(x2 * w3l_ref[0][None, :], axis=1)
    y3_ref[n:] = jnp.zeros((n_pad - n,), jnp.float32)
    r3_ref[...] = jnp.sum(x2 * w3r_ref[0][None, :], axis=1) + b3_ref[0]


def _k4_body(n, p_ref, inv_ref, r3_ref, o_ref):
    agg = p_ref[0, :n] + p_ref[1, :n]
    o_ref[...] = agg * inv_ref[...] + r3_ref[...]


def kernel(x, edge_index, W1l, W1r, b1, g1, be1, W2l, W2r, b2, g2, be2,
           W3l, W3r, b3):
    n, d_in = x.shape
    d_h = W1l.shape[0]
    e = edge_index.shape[1]
    assert e % CH == 0
    nchunk = e // CH
    n_pad = -(-(n + 1) // (NS * 8)) * (NS * 8)
    ei = edge_index.reshape(2, nchunk, CH)

    seg_d = _seg_kernel(n_pad, nchunk, d_h, True)
    seg = _seg_kernel(n_pad, nchunk, d_h, False)
    seg1 = _seg1_kernel(n_pad, nchunk)

    f32 = jnp.float32
    k1a = pl.pallas_call(
        functools.partial(_k1a_body, n, n_pad),
        out_shape=jax.ShapeDtypeStruct((n_pad, d_h), f32))
    y1 = k1a(x, W1l.T)
    k1b = pl.pallas_call(
        _k1b_body, out_shape=jax.ShapeDtypeStruct((n, d_h), f32))
    r1 = k1b(x, W1r.T, b1)

    part1, degp = seg_d(y1, ei)
    degp = degp.reshape(NC, n_pad)

    k2 = pl.pallas_call(
        functools.partial(_k2_body, n, n_pad),
        out_shape=(jax.ShapeDtypeStruct((n, d_h), f32),
                   jax.ShapeDtypeStruct((n_pad, d_h), f32),
                   jax.ShapeDtypeStruct((n,), f32)))
    x1, y2, inv = k2(part1, degp, r1, g1, be1, W2l.T)

    k2c = pl.pallas_call(
        _k2c_body, out_shape=jax.ShapeDtypeStruct((n, d_h), f32))
    r2 = k2c(x1, W2r.T, b2)

    part2 = seg(y2, ei)[0]

    k3 = pl.pallas_call(
        functools.partial(_k3_body, n, n_pad),
        out_shape=(jax.ShapeDtypeStruct((n_pad,), f32),
                   jax.ShapeDtypeStruct((n,), f32)))
    y3, r3 = k3(part2, inv, x1, r2, g2, be2, W3l, W3r, b3)

    part3 = seg1(y3, ei).reshape(NC, n_pad)

    k4 = pl.pallas_call(
        functools.partial(_k4_body, n),
        out_shape=jax.ShapeDtypeStruct((n,), f32))
    return k4(part3, inv, r3)


# speedup vs baseline: 1.0456x; 1.0456x over previous
import functools

import jax
import jax.numpy as jnp
from jax import lax
from jax.experimental import pallas as pl
from jax.experimental.pallas import tpu as pltpu
from jax.experimental.pallas import tpu_sc as plsc

NC = 2
NS = 16
NW = NC * NS
CH = 128
NBUF = 3

_SC_PARAMS = pltpu.CompilerParams(use_tc_tiling_on_sc=False)


def _zchunks(zrows):
    out = []
    off = 0
    while off < zrows:
        out.append((off, min(CH, zrows - off)))
        off += CH
    return out


def _fill(ref, nrows, d, vec16):
    def fi(i, carry):
        for jj in range(d // 16):
            ref[i, pl.ds(jj * 16, 16)] = vec16
        return carry
    lax.fori_loop(0, nrows, fi, 0)


def _seg_kernel(n_pad, nchunk, d, with_deg):
    zrows = n_pad // NS
    base = nchunk // NW
    left = nchunk - base * NW
    ngroups = base // NBUF
    assert base == ngroups * NBUF
    zck = _zchunks(zrows)
    mesh = plsc.VectorSubcoreMesh(core_axis_name="c", subcore_axis_name="s")

    out_type = [jax.ShapeDtypeStruct((NC, n_pad, d), jnp.float32)]
    scratch = [
        pltpu.VMEM_SHARED((n_pad, d), jnp.float32),
        pltpu.VMEM_SHARED((n_pad, d), jnp.float32),
        pltpu.VMEM((base + 1, CH), jnp.int32),
        pltpu.VMEM((base + 1, CH), jnp.int32),
    ]
    scratch += [pltpu.VMEM((CH, d), jnp.float32) for _ in range(NBUF)]
    scratch += [pltpu.SemaphoreType.DMA for _ in range(NBUF)]
    if with_deg:
        out_type.append(jax.ShapeDtypeStruct((NC * n_pad,), jnp.float32))
        scratch.append(pltpu.VMEM_SHARED((n_pad,), jnp.float32))
        scratch.append(pltpu.VMEM((CH,), jnp.float32))
        scratch.append(pltpu.VMEM((zrows,), jnp.float32))
        scratch.append(pltpu.SemaphoreType.DMA)

    def body(y_h, ei_h, part_h, *rest):
        if with_deg:
            deg_h = rest[0]
            rest = rest[1:]
        ycp, acc, si2, di2 = rest[:4]
        rows = rest[4:4 + NBUF]
        gsem = rest[4 + NBUF:4 + 2 * NBUF]
        if with_deg:
            dacc, ones, dstg, dsem = rest[4 + 2 * NBUF:]
        c = lax.axis_index("c")
        s = lax.axis_index("s")
        wid = s * NC + c
        r0 = s * zrows
        row0 = wid * base
        xrow = jnp.minimum(NW * base + wid, nchunk - 1)
        pltpu.async_copy(ei_h.at[0, pl.ds(row0, base)],
                         si2.at[pl.ds(0, base)], gsem[2])
        pltpu.async_copy(ei_h.at[1, pl.ds(row0, base)],
                         di2.at[pl.ds(0, base)], gsem[2])
        pltpu.async_copy(ei_h.at[0, xrow], si2.at[base], gsem[2])
        pltpu.async_copy(ei_h.at[1, xrow], di2.at[base], gsem[2])
        _fill(rows[1], CH, d, jnp.zeros((16,), jnp.float32))
        pbuf = (rows[0], rows[2])
        pltpu.async_copy(y_h.at[pl.ds(r0 + zck[0][0], zck[0][1])],
                         pbuf[0].at[pl.ds(0, zck[0][1])], gsem[0])
        for k, (zo, zs) in enumerate(zck):
            cur = pbuf[k % 2]
            pltpu.make_async_copy(y_h.at[pl.ds(r0 + zo, zs)],
                                  cur.at[pl.ds(0, zs)], gsem[k % 2]).wait()
            if k + 1 < len(zck):
                zo2, zs2 = zck[k + 1]
                pltpu.async_copy(y_h.at[pl.ds(r0 + zo2, zs2)],
                                 pbuf[(k + 1) % 2].at[pl.ds(0, zs2)],
                                 gsem[(k + 1) % 2])
            pltpu.sync_copy(cur.at[pl.ds(0, zs)],
                            ycp.at[pl.ds(r0 + zo, zs)])
            pltpu.sync_copy(rows[1].at[pl.ds(0, zs)],
                            acc.at[pl.ds(r0 + zo, zs)])
        pltpu.make_async_copy(ei_h.at[0, pl.ds(row0, base)],
                              si2.at[pl.ds(0, base)], gsem[2]).wait()
        pltpu.make_async_copy(ei_h.at[1, pl.ds(row0, base)],
                              di2.at[pl.ds(0, base)], gsem[2]).wait()
        pltpu.make_async_copy(ei_h.at[0, xrow], si2.at[base], gsem[2]).wait()
        pltpu.make_async_copy(ei_h.at[1, xrow], di2.at[base], gsem[2]).wait()
        if with_deg:
            for i in range(CH // 16):
                ones[pl.ds(i * 16, 16)] = jnp.ones((16,), jnp.float32)
            zd = jnp.zeros((16,), jnp.float32)
            for i in range(zrows // 16):
                dstg[pl.ds(i * 16, 16)] = zd
            dstg[pl.ds(zrows - 16, 16)] = zd
            pltpu.sync_copy(dstg, dacc.at[pl.ds(r0, zrows)])
        plsc.subcore_barrier()

        pltpu.async_copy(ycp.at[si2.at[0]], rows[0], gsem[0])
        pltpu.async_copy(ycp.at[si2.at[1]], rows[1], gsem[1])

        def deg_wait():
            pltpu.make_async_copy(ones, dacc.at[di2.at[0]], dsem).wait()

        def group(g, carry):
            for b in range(NBUF):
                j = g * NBUF + b
                q = (b + 2) % NBUF
                pltpu.make_async_copy(ycp.at[si2.at[b]], rows[b],
                                      gsem[b]).wait()
                if b == 0:
                    pltpu.async_copy(ycp.at[si2.at[j + 2]], rows[q], gsem[q])
                else:
                    @pl.when(g < ngroups - 1)
                    def _():
                        pltpu.async_copy(ycp.at[si2.at[j + 2]], rows[q],
                                         gsem[q])
                pltpu.sync_copy(rows[b], acc.at[di2.at[j]], add=True)
                if with_deg:
                    pltpu.async_copy(ones, dacc.at[di2.at[j]], dsem,
                                     add=True)

                    @pl.when(g >= 2)
                    def _():
                        deg_wait()
            return carry

        lax.fori_loop(0, ngroups, group, 0)
        if with_deg:
            for _ in range(2 * NBUF):
                deg_wait()

        @pl.when(wid < left)
        def _():
            pltpu.sync_copy(ycp.at[si2.at[base]], rows[0])
            pltpu.sync_copy(rows[0], acc.at[di2.at[base]], add=True)
            if with_deg:
                pltpu.sync_copy(ones, dacc.at[di2.at[base]], add=True)

        plsc.subcore_barrier()
        for zo, zs in zck:
            pltpu.sync_copy(acc.at[pl.ds(r0 + zo, zs)],
                            rows[0].at[pl.ds(0, zs)])
            pltpu.sync_copy(rows[0].at[pl.ds(0, zs)],
                            part_h.at[c, pl.ds(r0 + zo, zs)])
        if with_deg:
            pltpu.sync_copy(dacc.at[pl.ds(r0, zrows)], dstg)
            pltpu.sync_copy(dstg, deg_h.at[pl.ds(c * n_pad + r0, zrows)])

    return pl.kernel(body, out_type=tuple(out_type), mesh=mesh,
                     scratch_types=scratch, compiler_params=_SC_PARAMS)


def _seg1_kernel(n_pad, nchunk):
    zrows = n_pad // NS
    base = nchunk // NW
    left = nchunk - base * NW
    ngroups = base // NBUF
    mesh = plsc.VectorSubcoreMesh(core_axis_name="c", subcore_axis_name="s")

    scratch = [
        pltpu.VMEM_SHARED((n_pad,), jnp.float32),
        pltpu.VMEM_SHARED((n_pad,), jnp.float32),
        pltpu.VMEM((base + 1, CH), jnp.int32),
        pltpu.VMEM((base + 1, CH), jnp.int32),
        pltpu.VMEM((zrows,), jnp.float32),
    ]
    scratch += [pltpu.VMEM((CH,), jnp.float32) for _ in range(NBUF)]
    scratch += [pltpu.SemaphoreType.DMA for _ in range(NBUF)]

    def body(y_h, ei_h, part_h, ycp, acc, si2, di2, stg, *rest):
        vals = rest[:NBUF]
        gsem = rest[NBUF:]
        c = lax.axis_index("c")
        s = lax.axis_index("s")
        wid = s * NC + c
        r0 = s * zrows
        row0 = wid * base
        xrow = jnp.minimum(NW * base + wid, nchunk - 1)
        pltpu.sync_copy(ei_h.at[0, pl.ds(row0, base)],
                        si2.at[pl.ds(0, base)])
        pltpu.sync_copy(ei_h.at[1, pl.ds(row0, base)],
                        di2.at[pl.ds(0, base)])
        pltpu.sync_copy(ei_h.at[0, xrow], si2.at[base])
        pltpu.sync_copy(ei_h.at[1, xrow], di2.at[base])
        pltpu.sync_copy(y_h.at[pl.ds(r0, zrows)], stg)
        pltpu.sync_copy(stg, ycp.at[pl.ds(r0, zrows)])
        zd = jnp.zeros((16,), jnp.float32)
        for i in range(zrows // 16):
            stg[pl.ds(i * 16, 16)] = zd
        stg[pl.ds(zrows - 16, 16)] = zd
        pltpu.sync_copy(stg, acc.at[pl.ds(r0, zrows)])
        plsc.subcore_barrier()

        pltpu.async_copy(ycp.at[si2.at[0]], vals[0], gsem[0])
        pltpu.async_copy(ycp.at[si2.at[1]], vals[1], gsem[1])

        def group(g, carry):
            for b in range(NBUF):
                j = g * NBUF + b
                q = (b + 2) % NBUF
                pltpu.make_async_copy(ycp.at[si2.at[b]], vals[b],
                                      gsem[b]).wait()
                if b == 0:
                    pltpu.async_copy(ycp.at[si2.at[j + 2]], vals[q], gsem[q])
                else:
                    @pl.when(g < ngroups - 1)
                    def _():
                        pltpu.async_copy(ycp.at[si2.at[j + 2]], vals[q],
                                         gsem[q])
                pltpu.sync_copy(vals[b], acc.at[di2.at[j]], add=True)
            return carry

        lax.fori_loop(0, ngroups, group, 0)

        @pl.when(wid < left)
        def _():
            pltpu.sync_copy(ycp.at[si2.at[base]], vals[0])
            pltpu.sync_copy(vals[0], acc.at[di2.at[base]], add=True)

        plsc.subcore_barrier()
        pltpu.sync_copy(acc.at[pl.ds(r0, zrows)], stg)
        pltpu.sync_copy(stg, part_h.at[pl.ds(c * n_pad + r0, zrows)])

    return pl.kernel(
        body,
        out_type=jax.ShapeDtypeStruct((NC * n_pad,), jnp.float32),
        mesh=mesh, scratch_types=scratch, compiler_params=_SC_PARAMS)


def _k1_body(n, n_pad, x_ref, wl_ref, wr_ref, b_ref, y_ref, r_ref):
    x = x_ref[...]
    y_ref[:n, :] = jnp.dot(x, wl_ref[...], preferred_element_type=jnp.float32)
    y_ref[n:, :] = jnp.zeros((n_pad - n, y_ref.shape[1]), jnp.float32)
    r_ref[...] = (jnp.dot(x, wr_ref[...], preferred_element_type=jnp.float32)
                  + b_ref[...][None, :])


def _k2_body(n, n_pad, p_ref, d_ref, r1_ref, g_ref, be_ref, wl_ref,
             x1_ref, y2_ref, inv_ref):
    agg = p_ref[0, :n, :] + p_ref[1, :n, :]
    deg = d_ref[0, :n] + d_ref[1, :n]
    inv = 1.0 / jnp.maximum(deg, 1.0)
    t = agg * inv[:, None] + r1_ref[...]
    mu = jnp.mean(t, axis=0)
    var = jnp.mean((t - mu[None, :]) ** 2, axis=0)
    xh = (t - mu[None, :]) * lax.rsqrt(var + 1e-5)[None, :]
    x1 = jnp.maximum(xh * g_ref[...][None, :] + be_ref[...][None, :], 0.0)
    x1_ref[...] = x1
    y2_ref[:n, :] = jnp.dot(x1, wl_ref[...],
                            preferred_element_type=jnp.float32)
    y2_ref[n:, :] = jnp.zeros((n_pad - n, y2_ref.shape[1]), jnp.float32)
    inv_ref[...] = inv


def _k3_body(n, n_pad, p_ref, inv_ref, x1_ref, wr_ref, b2_ref, g_ref, be_ref,
             w3l_ref, w3r_ref, b3_ref, y3_ref, r3_ref):
    agg = p_ref[0, :n, :] + p_ref[1, :n, :]
    inv = inv_ref[...]
    x1 = x1_ref[...]
    r2 = (jnp.dot(x1, wr_ref[...], preferred_element_type=jnp.float32)
          + b2_ref[...][None, :])
    t = agg * inv[:, None] + r2
    mu = jnp.mean(t, axis=0)
    var = jnp.mean((t - mu[None, :]) ** 2, axis=0)
    xh = (t - mu[None, :]) * lax.rsqrt(var + 1e-5)[None, :]
    x2 = jnp.maximum(xh * g_ref[...][None, :] + be_ref[...][None, :], 0.0)
    x2 = x2 + x1
    y3_ref[:n] = jnp.sum(x2 * w3l_ref[0][None, :], axis=1)
    y3_ref[n:] = jnp.zeros((n_pad - n,), jnp.float32)
    r3_ref[...] = jnp.sum(x2 * w3r_ref[0][None, :], axis=1) + b3_ref[0]


def _k4_body(n, p_ref, inv_ref, r3_ref, o_ref):
    agg = p_ref[0, :n] + p_ref[1, :n]
    o_ref[...] = agg * inv_ref[...] + r3_ref[...]


def kernel(x, edge_index, W1l, W1r, b1, g1, be1, W2l, W2r, b2, g2, be2,
           W3l, W3r, b3):
    n, d_in = x.shape
    d_h = W1l.shape[0]
    e = edge_index.shape[1]
    assert e % CH == 0
    nchunk = e // CH
    n_pad = -(-(n + 1) // (NS * 8)) * (NS * 8)
    ei = edge_index.reshape(2, nchunk, CH)

    seg_d = _seg_kernel(n_pad, nchunk, d_h, True)
    seg = _seg_kernel(n_pad, nchunk, d_h, False)
    seg1 = _seg1_kernel(n_pad, nchunk)

    f32 = jnp.float32
    k1 = pl.pallas_call(
        functools.partial(_k1_body, n, n_pad),
        out_shape=(jax.ShapeDtypeStruct((n_pad, d_h), f32),
                   jax.ShapeDtypeStruct((n, d_h), f32)))
    y1, r1 = k1(x, W1l.T, W1r.T, b1)

    part1, degp = seg_d(y1, ei)
    degp = degp.reshape(NC, n_pad)

    k2 = pl.pallas_call(
        functools.partial(_k2_body, n, n_pad),
        out_shape=(jax.ShapeDtypeStruct((n, d_h), f32),
                   jax.ShapeDtypeStruct((n_pad, d_h), f32),
                   jax.ShapeDtypeStruct((n,), f32)))
    x1, y2, inv = k2(part1, degp, r1, g1, be1, W2l.T)

    part2 = seg(y2, ei)[0]

    k3 = pl.pallas_call(
        functools.partial(_k3_body, n, n_pad),
        out_shape=(jax.ShapeDtypeStruct((n_pad,), f32),
                   jax.ShapeDtypeStruct((n,), f32)))
    y3, r3 = k3(part2, inv, x1, W2r.T, b2, g2, be2, W3l, W3r, b3)

    part3 = seg1(y3, ei).reshape(NC, n_pad)

    k4 = pl.pallas_call(
        functools.partial(_k4_body, n),
        out_shape=jax.ShapeDtypeStruct((n,), f32))
    return k4(part3, inv, r3)


# speedup vs baseline: 1.0578x; 1.0117x over previous
import functools

import jax
import jax.numpy as jnp
from jax import lax
from jax.experimental import pallas as pl
from jax.experimental.pallas import tpu as pltpu
from jax.experimental.pallas import tpu_sc as plsc

NC = 2
NS = 16
NW = NC * NS
CH = 128
NBUF = 3

_SC_PARAMS = pltpu.CompilerParams(use_tc_tiling_on_sc=False)


def _zchunks(zrows):
    out = []
    off = 0
    while off < zrows:
        out.append((off, min(CH, zrows - off)))
        off += CH
    return out


def _fill(ref, nrows, d, vec16):
    def fi(i, carry):
        for jj in range(d // 16):
            ref[i, pl.ds(jj * 16, 16)] = vec16
        return carry
    lax.fori_loop(0, nrows, fi, 0)


def _seg_kernel(n_pad, nchunk, d, with_deg):
    zrows = n_pad // NS
    base = nchunk // NW
    left = nchunk - base * NW
    ngroups = base // NBUF
    assert base == ngroups * NBUF
    zck = _zchunks(zrows)
    mesh = plsc.VectorSubcoreMesh(core_axis_name="c", subcore_axis_name="s")

    out_type = [jax.ShapeDtypeStruct((NC, n_pad, d), jnp.float32)]
    scratch = [
        pltpu.VMEM_SHARED((n_pad, d), jnp.float32),
        pltpu.VMEM_SHARED((n_pad, d), jnp.float32),
        pltpu.VMEM((base + 1, CH), jnp.int32),
        pltpu.VMEM((base + 1, CH), jnp.int32),
    ]
    scratch += [pltpu.VMEM((CH, d), jnp.float32) for _ in range(NBUF)]
    scratch += [pltpu.SemaphoreType.DMA for _ in range(NBUF)]
    if with_deg:
        out_type.append(jax.ShapeDtypeStruct((NC * n_pad,), jnp.float32))
        scratch.append(pltpu.VMEM_SHARED((n_pad,), jnp.float32))
        scratch.append(pltpu.VMEM((CH,), jnp.float32))
        scratch.append(pltpu.VMEM((zrows,), jnp.float32))
        scratch.append(pltpu.SemaphoreType.DMA)

    def body(y_h, ei_h, part_h, *rest):
        if with_deg:
            deg_h = rest[0]
            rest = rest[1:]
        ycp, acc, si2, di2 = rest[:4]
        rows = rest[4:4 + NBUF]
        gsem = rest[4 + NBUF:4 + 2 * NBUF]
        if with_deg:
            dacc, ones, dstg, dsem = rest[4 + 2 * NBUF:]
        c = lax.axis_index("c")
        s = lax.axis_index("s")
        wid = s * NC + c
        r0 = s * zrows
        row0 = wid * base
        xrow = jnp.minimum(NW * base + wid, nchunk - 1)
        pltpu.async_copy(ei_h.at[0, pl.ds(row0, base)],
                         si2.at[pl.ds(0, base)], gsem[2])
        pltpu.async_copy(ei_h.at[1, pl.ds(row0, base)],
                         di2.at[pl.ds(0, base)], gsem[2])
        pltpu.async_copy(ei_h.at[0, xrow], si2.at[base], gsem[2])
        pltpu.async_copy(ei_h.at[1, xrow], di2.at[base], gsem[2])
        _fill(rows[1], CH, d, jnp.zeros((16,), jnp.float32))
        pbuf = (rows[0], rows[2])
        pltpu.async_copy(y_h.at[pl.ds(r0 + zck[0][0], zck[0][1])],
                         pbuf[0].at[pl.ds(0, zck[0][1])], gsem[0])
        for k, (zo, zs) in enumerate(zck):
            cur = pbuf[k % 2]
            pltpu.make_async_copy(y_h.at[pl.ds(r0 + zo, zs)],
                                  cur.at[pl.ds(0, zs)], gsem[k % 2]).wait()
            if k + 1 < len(zck):
                zo2, zs2 = zck[k + 1]
                pltpu.async_copy(y_h.at[pl.ds(r0 + zo2, zs2)],
                                 pbuf[(k + 1) % 2].at[pl.ds(0, zs2)],
                                 gsem[(k + 1) % 2])
            pltpu.sync_copy(cur.at[pl.ds(0, zs)],
                            ycp.at[pl.ds(r0 + zo, zs)])
            pltpu.sync_copy(rows[1].at[pl.ds(0, zs)],
                            acc.at[pl.ds(r0 + zo, zs)])
        pltpu.make_async_copy(ei_h.at[0, pl.ds(row0, base)],
                              si2.at[pl.ds(0, base)], gsem[2]).wait()
        pltpu.make_async_copy(ei_h.at[1, pl.ds(row0, base)],
                              di2.at[pl.ds(0, base)], gsem[2]).wait()
        pltpu.make_async_copy(ei_h.at[0, xrow], si2.at[base], gsem[2]).wait()
        pltpu.make_async_copy(ei_h.at[1, xrow], di2.at[base], gsem[2]).wait()
        if with_deg:
            for i in range(CH // 16):
                ones[pl.ds(i * 16, 16)] = jnp.ones((16,), jnp.float32)
            zd = jnp.zeros((16,), jnp.float32)
            for i in range(zrows // 16):
                dstg[pl.ds(i * 16, 16)] = zd
            dstg[pl.ds(zrows - 16, 16)] = zd
            pltpu.sync_copy(dstg, dacc.at[pl.ds(r0, zrows)])
        plsc.subcore_barrier()

        pltpu.async_copy(ycp.at[si2.at[0]], rows[0], gsem[0])
        pltpu.async_copy(ycp.at[si2.at[1]], rows[1], gsem[1])

        def deg_wait():
            pltpu.make_async_copy(ones, dacc.at[di2.at[0]], dsem).wait()

        def group(g, carry):
            for b in range(NBUF):
                j = g * NBUF + b
                q = (b + 2) % NBUF
                pltpu.make_async_copy(ycp.at[si2.at[b]], rows[b],
                                      gsem[b]).wait()
                if b == 0:
                    pltpu.async_copy(ycp.at[si2.at[j + 2]], rows[q], gsem[q])
                else:
                    @pl.when(g < ngroups - 1)
                    def _():
                        pltpu.async_copy(ycp.at[si2.at[j + 2]], rows[q],
                                         gsem[q])
                pltpu.sync_copy(rows[b], acc.at[di2.at[j]], add=True)
                if with_deg:
                    pltpu.async_copy(ones, dacc.at[di2.at[j]], dsem,
                                     add=True)

                    @pl.when(g >= 2)
                    def _():
                        deg_wait()
            return carry

        lax.fori_loop(0, ngroups, group, 0)
        if with_deg:
            for _ in range(2 * NBUF):
                deg_wait()

        @pl.when(wid < left)
        def _():
            pltpu.sync_copy(ycp.at[si2.at[base]], rows[0])
            pltpu.sync_copy(rows[0], acc.at[di2.at[base]], add=True)
            if with_deg:
                pltpu.sync_copy(ones, dacc.at[di2.at[base]], add=True)

        plsc.subcore_barrier()
        for zo, zs in zck:
            pltpu.sync_copy(acc.at[pl.ds(r0 + zo, zs)],
                            rows[0].at[pl.ds(0, zs)])
            pltpu.sync_copy(rows[0].at[pl.ds(0, zs)],
                            part_h.at[c, pl.ds(r0 + zo, zs)])
        if with_deg:
            pltpu.sync_copy(dacc.at[pl.ds(r0, zrows)], dstg)
            pltpu.sync_copy(dstg, deg_h.at[pl.ds(c * n_pad + r0, zrows)])

    return pl.kernel(body, out_type=tuple(out_type), mesh=mesh,
                     scratch_types=scratch, compiler_params=_SC_PARAMS)


def _seg1_kernel(n_pad, nchunk):
    zrows = n_pad // NS
    base = nchunk // NW
    left = nchunk - base * NW
    ngroups = base // NBUF
    mesh = plsc.VectorSubcoreMesh(core_axis_name="c", subcore_axis_name="s")

    scratch = [
        pltpu.VMEM_SHARED((n_pad,), jnp.float32),
        pltpu.VMEM_SHARED((n_pad,), jnp.float32),
        pltpu.VMEM((base + 1, CH), jnp.int32),
        pltpu.VMEM((base + 1, CH), jnp.int32),
        pltpu.VMEM((zrows,), jnp.float32),
    ]
    scratch += [pltpu.VMEM((CH,), jnp.float32) for _ in range(NBUF)]
    scratch += [pltpu.SemaphoreType.DMA for _ in range(NBUF)]

    def body(y_h, ei_h, part_h, ycp, acc, si2, di2, stg, *rest):
        vals = rest[:NBUF]
        gsem = rest[NBUF:]
        c = lax.axis_index("c")
        s = lax.axis_index("s")
        wid = s * NC + c
        r0 = s * zrows
        row0 = wid * base
        xrow = jnp.minimum(NW * base + wid, nchunk - 1)
        pltpu.async_copy(ei_h.at[0, pl.ds(row0, base)],
                         si2.at[pl.ds(0, base)], gsem[2])
        pltpu.async_copy(ei_h.at[1, pl.ds(row0, base)],
                         di2.at[pl.ds(0, base)], gsem[2])
        pltpu.async_copy(ei_h.at[0, xrow], si2.at[base], gsem[2])
        pltpu.async_copy(ei_h.at[1, xrow], di2.at[base], gsem[2])
        pltpu.sync_copy(y_h.at[pl.ds(r0, zrows)], stg)
        pltpu.sync_copy(stg, ycp.at[pl.ds(r0, zrows)])
        zd = jnp.zeros((16,), jnp.float32)
        for i in range(zrows // 16):
            stg[pl.ds(i * 16, 16)] = zd
        stg[pl.ds(zrows - 16, 16)] = zd
        pltpu.sync_copy(stg, acc.at[pl.ds(r0, zrows)])
        pltpu.make_async_copy(ei_h.at[0, pl.ds(row0, base)],
                              si2.at[pl.ds(0, base)], gsem[2]).wait()
        pltpu.make_async_copy(ei_h.at[1, pl.ds(row0, base)],
                              di2.at[pl.ds(0, base)], gsem[2]).wait()
        pltpu.make_async_copy(ei_h.at[0, xrow], si2.at[base], gsem[2]).wait()
        pltpu.make_async_copy(ei_h.at[1, xrow], di2.at[base], gsem[2]).wait()
        plsc.subcore_barrier()

        pltpu.async_copy(ycp.at[si2.at[0]], vals[0], gsem[0])
        pltpu.async_copy(ycp.at[si2.at[1]], vals[1], gsem[1])

        def group(g, carry):
            for b in range(NBUF):
                j = g * NBUF + b
                q = (b + 2) % NBUF
                pltpu.make_async_copy(ycp.at[si2.at[b]], vals[b],
                                      gsem[b]).wait()
                if b == 0:
                    pltpu.async_copy(ycp.at[si2.at[j + 2]], vals[q], gsem[q])
                else:
                    @pl.when(g < ngroups - 1)
                    def _():
                        pltpu.async_copy(ycp.at[si2.at[j + 2]], vals[q],
                                         gsem[q])
                pltpu.sync_copy(vals[b], acc.at[di2.at[j]], add=True)
            return carry

        lax.fori_loop(0, ngroups, group, 0)

        @pl.when(wid < left)
        def _():
            pltpu.sync_copy(ycp.at[si2.at[base]], vals[0])
            pltpu.sync_copy(vals[0], acc.at[di2.at[base]], add=True)

        plsc.subcore_barrier()
        pltpu.sync_copy(acc.at[pl.ds(r0, zrows)], stg)
        pltpu.sync_copy(stg, part_h.at[pl.ds(c * n_pad + r0, zrows)])

    return pl.kernel(
        body,
        out_type=jax.ShapeDtypeStruct((NC * n_pad,), jnp.float32),
        mesh=mesh, scratch_types=scratch, compiler_params=_SC_PARAMS)


def _k1_body(n, n_pad, x_ref, wl_ref, wr_ref, b_ref, y_ref, r_ref):
    x = x_ref[...]
    y_ref[:n, :] = jnp.dot(x, wl_ref[...], preferred_element_type=jnp.float32)
    y_ref[n:, :] = jnp.zeros((n_pad - n, y_ref.shape[1]), jnp.float32)
    r_ref[...] = (jnp.dot(x, wr_ref[...], preferred_element_type=jnp.float32)
                  + b_ref[...][None, :])


def _k2_body(n, n_pad, p_ref, d_ref, r1_ref, g_ref, be_ref, wl_ref,
             x1_ref, y2_ref, inv_ref):
    agg = p_ref[0, :n, :] + p_ref[1, :n, :]
    deg = d_ref[0, :n] + d_ref[1, :n]
    inv = 1.0 / jnp.maximum(deg, 1.0)
    t = agg * inv[:, None] + r1_ref[...]
    mu = jnp.mean(t, axis=0)
    var = jnp.mean((t - mu[None, :]) ** 2, axis=0)
    xh = (t - mu[None, :]) * lax.rsqrt(var + 1e-5)[None, :]
    x1 = jnp.maximum(xh * g_ref[...][None, :] + be_ref[...][None, :], 0.0)
    x1_ref[...] = x1
    y2_ref[:n, :] = jnp.dot(x1, wl_ref[...],
                            preferred_element_type=jnp.float32)
    y2_ref[n:, :] = jnp.zeros((n_pad - n, y2_ref.shape[1]), jnp.float32)
    inv_ref[...] = inv


def _k3_body(n, n_pad, p_ref, inv_ref, x1_ref, wr_ref, b2_ref, g_ref, be_ref,
             w3l_ref, w3r_ref, b3_ref, y3_ref, r3_ref):
    agg = p_ref[0, :n, :] + p_ref[1, :n, :]
    inv = inv_ref[...]
    x1 = x1_ref[...]
    r2 = (jnp.dot(x1, wr_ref[...], preferred_element_type=jnp.float32)
          + b2_ref[...][None, :])
    t = agg * inv[:, None] + r2
    mu = jnp.mean(t, axis=0)
    var = jnp.mean((t - mu[None, :]) ** 2, axis=0)
    xh = (t - mu[None, :]) * lax.rsqrt(var + 1e-5)[None, :]
    x2 = jnp.maximum(xh * g_ref[...][None, :] + be_ref[...][None, :], 0.0)
    x2 = x2 + x1
    y3_ref[:n] = jnp.sum(x2 * w3l_ref[0][None, :], axis=1)
    y3_ref[n:] = jnp.zeros((n_pad - n,), jnp.float32)
    r3_ref[...] = jnp.sum(x2 * w3r_ref[0][None, :], axis=1) + b3_ref[0]


def _k4_body(n, p_ref, inv_ref, r3_ref, o_ref):
    agg = p_ref[0, :n] + p_ref[1, :n]
    o_ref[...] = agg * inv_ref[...] + r3_ref[...]


def kernel(x, edge_index, W1l, W1r, b1, g1, be1, W2l, W2r, b2, g2, be2,
           W3l, W3r, b3):
    n, d_in = x.shape
    d_h = W1l.shape[0]
    e = edge_index.shape[1]
    assert e % CH == 0
    nchunk = e // CH
    n_pad = -(-(n + 1) // (NS * 8)) * (NS * 8)
    ei = edge_index.reshape(2, nchunk, CH)

    seg_d = _seg_kernel(n_pad, nchunk, d_h, True)
    seg = _seg_kernel(n_pad, nchunk, d_h, False)
    seg1 = _seg1_kernel(n_pad, nchunk)

    f32 = jnp.float32
    k1 = pl.pallas_call(
        functools.partial(_k1_body, n, n_pad),
        out_shape=(jax.ShapeDtypeStruct((n_pad, d_h), f32),
                   jax.ShapeDtypeStruct((n, d_h), f32)))
    y1, r1 = k1(x, W1l.T, W1r.T, b1)

    part1, degp = seg_d(y1, ei)
    degp = degp.reshape(NC, n_pad)

    k2 = pl.pallas_call(
        functools.partial(_k2_body, n, n_pad),
        out_shape=(jax.ShapeDtypeStruct((n, d_h), f32),
                   jax.ShapeDtypeStruct((n_pad, d_h), f32),
                   jax.ShapeDtypeStruct((n,), f32)))
    x1, y2, inv = k2(part1, degp, r1, g1, be1, W2l.T)

    part2 = seg(y2, ei)[0]

    k3 = pl.pallas_call(
        functools.partial(_k3_body, n, n_pad),
        out_shape=(jax.ShapeDtypeStruct((n_pad,), f32),
                   jax.ShapeDtypeStruct((n,), f32)))
    y3, r3 = k3(part2, inv, x1, W2r.T, b2, g2, be2, W3l, W3r, b3)

    part3 = seg1(y3, ei).reshape(NC, n_pad)

    k4 = pl.pallas_call(
        functools.partial(_k4_body, n),
        out_shape=jax.ShapeDtypeStruct((n,), f32))
    return k4(part3, inv, r3)
